# baseline (speedup 1.0000x reference)
import functools

import jax
import jax.numpy as jnp
from jax import lax
from jax.experimental import pallas as pl
from jax.experimental.pallas import tpu as pltpu
from jax.experimental.pallas import tpu_sc as plsc

NUM_TOKENS = 8192
D_IN = 2048
D_OUT = 2048
RANK = 16
NUM_EXPERTS = 16
SCALING = 32 / float(RANK)
ER = NUM_EXPERTS * RANK

TOKEN_BLOCK = 512

NC, NS = 2, 16
NW = NC * NS
H_TOK = NUM_TOKENS // 2
TPW = H_TOK // NW
QTR = TPW // 2


def _inter_kernel(x_ref, a_ref, o_ref):
    o_ref[...] = jax.lax.dot_general(
        x_ref[...], a_ref[...], (((1,), (1,)), ((), ())),
        preferred_element_type=jnp.float32)


def _sc_mask_body(inter_hbm, eid_hbm, out_hbm,
                  eid_v, in0, in1, out_v, sem0, sem1, half):
    wid = lax.axis_index("s") * NC + lax.axis_index("c")
    base = half * H_TOK + wid * TPW
    obase = wid * TPW

    pltpu.sync_copy(eid_hbm.at[pl.ds(base, TPW)], eid_v)

    ins = (in0, in1)
    sems = (sem0, sem1)

    def fetch(q, buf):
        return pltpu.async_copy(
            inter_hbm.at[pl.ds(base + q * QTR, QTR), :],
            ins[buf], sems[buf])

    cps = [fetch(0, 0), fetch(1, 1)]

    zero16 = jnp.zeros((16,), jnp.float32)

    def zr(r, carry):
        for k in range(ER // 16):
            out_v[r, pl.ds(k * 16, 16)] = zero16
        return carry

    lax.fori_loop(0, TPW, zr, None)

    for q in range(2):
        cps[q].wait()
        in_v = ins[q]

        def put(g, carry, q=q, in_v=in_v):
            e16 = eid_v[pl.ds(q * QTR + g * 16, 16)]
            for j in range(16):
                t = g * 16 + j
                off = e16[j] * RANK
                out_v[q * QTR + t, pl.ds(off, RANK)] = (
                    in_v[t, pl.ds(off, RANK)])
            return carry

        lax.fori_loop(0, QTR // 16, put, None)

    pltpu.sync_copy(out_v, out_hbm.at[pl.ds(obase, TPW), :])


def _sc_mask_half(inter_all, eids, half):
    return pl.kernel(
        functools.partial(_sc_mask_body, half=half),
        out_type=jax.ShapeDtypeStruct((H_TOK, ER), jnp.float32),
        mesh=plsc.VectorSubcoreMesh(core_axis_name="c", subcore_axis_name="s"),
        scratch_types=[
            pltpu.VMEM((TPW,), jnp.int32),
            pltpu.VMEM((QTR, ER), jnp.float32),
            pltpu.VMEM((QTR, ER), jnp.float32),
            pltpu.VMEM((TPW, ER), jnp.float32),
            pltpu.SemaphoreType.DMA,
            pltpu.SemaphoreType.DMA,
        ],
        name=f"sc_mask_h{half}",
    )(inter_all, eids)


def _final_kernel(x_ref, m_ref, w_ref, b_ref, bt_ref, o_ref):
    base = jax.lax.dot_general(
        x_ref[...], w_ref[...], (((1,), (1,)), ((), ())),
        preferred_element_type=jnp.float32)
    delta = jax.lax.dot_general(
        m_ref[...], bt_ref[...], (((1,), (0,)), ((), ())),
        preferred_element_type=jnp.float32)
    o_ref[...] = base + b_ref[...] + SCALING * delta


def _final_kernel_h1(prev_ref, x_ref, m_ref, w_ref, b_ref, bt_ref, o_ref):
    del prev_ref
    _final_kernel(x_ref, m_ref, w_ref, b_ref, bt_ref, o_ref)


def _final_half(x, masked_h, W, b2, bt, half, prev=None):
    grid = (H_TOK // TOKEN_BLOCK,)
    blk = half * (H_TOK // TOKEN_BLOCK)
    in_specs = [
        pl.BlockSpec((TOKEN_BLOCK, D_IN), lambda i: (i + blk, 0)),
        pl.BlockSpec((TOKEN_BLOCK, ER), lambda i: (i, 0)),
        pl.BlockSpec((D_OUT, D_IN), lambda i: (0, 0)),
        pl.BlockSpec((1, D_OUT), lambda i: (0, 0)),
        pl.BlockSpec((ER, D_OUT), lambda i: (0, 0)),
    ]
    args = [x, masked_h, W, b2, bt]
    body = _final_kernel
    aliases = {}
    if prev is not None:
        in_specs = [pl.BlockSpec(memory_space=pltpu.MemorySpace.HBM)] + in_specs
        args = [prev] + args
        body = _final_kernel_h1
        aliases = {0: 0}
    return pl.pallas_call(
        body,
        grid=grid,
        in_specs=in_specs,
        out_specs=pl.BlockSpec((TOKEN_BLOCK, D_OUT), lambda i: (i + blk, 0)),
        out_shape=jax.ShapeDtypeStruct((NUM_TOKENS, D_OUT), jnp.float32),
        input_output_aliases=aliases,
    )(*args)


@functools.partial(jax.jit, static_argnames=())
def kernel(x, expert_ids, W, b, lora_a, lora_b):
    n_tokens = x.shape[0]
    eids = expert_ids.astype(jnp.int32)
    a_flat = lora_a.reshape(ER, D_IN)
    bt = lora_b.transpose(0, 2, 1).reshape(ER, D_OUT)
    b2 = b.reshape(1, D_OUT)

    inter_all = pl.pallas_call(
        _inter_kernel,
        grid=(n_tokens // 2048,),
        in_specs=[
            pl.BlockSpec((2048, D_IN), lambda i: (i, 0)),
            pl.BlockSpec((ER, D_IN), lambda i: (0, 0)),
        ],
        out_specs=pl.BlockSpec((2048, ER), lambda i: (i, 0)),
        out_shape=jax.ShapeDtypeStruct((n_tokens, ER), jnp.float32),
    )(x, a_flat)

    masked_h0 = _sc_mask_half(inter_all, eids, 0)
    masked_h1 = _sc_mask_half(inter_all, eids, 1)

    out0 = _final_half(x, masked_h0, W, b2, bt, half=0)
    out = _final_half(x, masked_h1, W, b2, bt, half=1, prev=out0)
    return out


# speedup vs baseline: 1.0300x; 1.0300x over previous
import functools

import jax
import jax.numpy as jnp
from jax import lax
from jax.experimental import pallas as pl
from jax.experimental.pallas import tpu as pltpu
from jax.experimental.pallas import tpu_sc as plsc

NUM_TOKENS = 8192
D_IN = 2048
D_OUT = 2048
RANK = 16
NUM_EXPERTS = 16
SCALING = 32 / float(RANK)
ER = NUM_EXPERTS * RANK

TOKEN_BLOCK = 512

NC, NS = 2, 16
NW = NC * NS
H_TOK = NUM_TOKENS // 2
HBLK = H_TOK // TOKEN_BLOCK
TPW = H_TOK // NW
QTR = TPW // 2


def _inter_kernel(x_ref, a_ref, o_ref):
    o_ref[...] = jax.lax.dot_general(
        x_ref[...], a_ref[...], (((1,), (1,)), ((), ())),
        preferred_element_type=jnp.float32)


def _inter_half(x, a_flat, half):
    return pl.pallas_call(
        _inter_kernel,
        grid=(4,),
        in_specs=[
            pl.BlockSpec((1024, D_IN), lambda i, h=half: (i + h * 4, 0)),
            pl.BlockSpec((ER, D_IN), lambda i: (0, 0)),
        ],
        out_specs=pl.BlockSpec((1024, ER), lambda i: (i, 0)),
        out_shape=jax.ShapeDtypeStruct((H_TOK, ER), jnp.float32),
    )(x, a_flat)


def _sc_mask_body(inter_hbm, eid_hbm, out_hbm,
                  eid_v, in0, in1, out_v, sem0, sem1, half):
    wid = lax.axis_index("s") * NC + lax.axis_index("c")
    base = wid * TPW
    ebase = half * H_TOK + wid * TPW

    pltpu.sync_copy(eid_hbm.at[pl.ds(ebase, TPW)], eid_v)

    ins = (in0, in1)
    sems = (sem0, sem1)

    def fetch(q, buf):
        return pltpu.async_copy(
            inter_hbm.at[pl.ds(base + q * QTR, QTR), :],
            ins[buf], sems[buf])

    cps = [fetch(0, 0), fetch(1, 1)]

    zero16 = jnp.zeros((16,), jnp.float32)

    def zr(r, carry):
        for k in range(ER // 16):
            out_v[r, pl.ds(k * 16, 16)] = zero16
        return carry

    lax.fori_loop(0, TPW, zr, None)

    for q in range(2):
        cps[q].wait()
        in_v = ins[q]

        def put(g, carry, q=q, in_v=in_v):
            e16 = eid_v[pl.ds(q * QTR + g * 16, 16)]
            for j in range(16):
                t = g * 16 + j
                off = e16[j] * RANK
                out_v[q * QTR + t, pl.ds(off, RANK)] = (
                    in_v[t, pl.ds(off, RANK)])
            return carry

        lax.fori_loop(0, QTR // 16, put, None)

    pltpu.sync_copy(out_v, out_hbm.at[pl.ds(base, TPW), :])


def _sc_mask_half(inter_h, eids, half):
    return pl.kernel(
        functools.partial(_sc_mask_body, half=half),
        out_type=jax.ShapeDtypeStruct((H_TOK, ER), jnp.float32),
        mesh=plsc.VectorSubcoreMesh(core_axis_name="c", subcore_axis_name="s"),
        scratch_types=[
            pltpu.VMEM((TPW,), jnp.int32),
            pltpu.VMEM((QTR, ER), jnp.float32),
            pltpu.VMEM((QTR, ER), jnp.float32),
            pltpu.VMEM((TPW, ER), jnp.float32),
            pltpu.SemaphoreType.DMA,
            pltpu.SemaphoreType.DMA,
        ],
        name=f"sc_mask_h{half}",
    )(inter_h, eids)


def _final_kernel(x_ref, m0_ref, m1_ref, w_ref, b_ref, bt_ref, o_ref):
    i = pl.program_id(0)
    base = jax.lax.dot_general(
        x_ref[...], w_ref[...], (((1,), (1,)), ((), ())),
        preferred_element_type=jnp.float32)
    m = jnp.where(i < HBLK, m0_ref[...], m1_ref[...])
    delta = jax.lax.dot_general(
        m, bt_ref[...], (((1,), (0,)), ((), ())),
        preferred_element_type=jnp.float32)
    o_ref[...] = base + b_ref[...] + SCALING * delta


@functools.partial(jax.jit, static_argnames=())
def kernel(x, expert_ids, W, b, lora_a, lora_b):
    n_tokens = x.shape[0]
    eids = expert_ids.astype(jnp.int32)
    a_flat = lora_a.reshape(ER, D_IN)
    bt = lora_b.transpose(0, 2, 1).reshape(ER, D_OUT)
    b2 = b.reshape(1, D_OUT)

    inter_h0 = _inter_half(x, a_flat, 0)
    inter_h1 = _inter_half(x, a_flat, 1)

    masked_h0 = _sc_mask_half(inter_h0, eids, 0)
    masked_h1 = _sc_mask_half(inter_h1, eids, 1)

    out = pl.pallas_call(
        _final_kernel,
        grid=(n_tokens // TOKEN_BLOCK,),
        in_specs=[
            pl.BlockSpec((TOKEN_BLOCK, D_IN), lambda i: (i, 0)),
            pl.BlockSpec((TOKEN_BLOCK, ER),
                         lambda i: (jnp.minimum(i, HBLK - 1), 0)),
            pl.BlockSpec((TOKEN_BLOCK, ER),
                         lambda i: (jnp.maximum(i - HBLK, 0), 0)),
            pl.BlockSpec((D_OUT, D_IN), lambda i: (0, 0)),
            pl.BlockSpec((1, D_OUT), lambda i: (0, 0)),
            pl.BlockSpec((ER, D_OUT), lambda i: (0, 0)),
        ],
        out_specs=pl.BlockSpec((TOKEN_BLOCK, D_OUT), lambda i: (i, 0)),
        out_shape=jax.ShapeDtypeStruct((n_tokens, D_OUT), jnp.float32),
    )(x, masked_h0, masked_h1, W, b2, bt)
    return out
